# Initial kernel scaffold; baseline (speedup 1.0000x reference)
import jax
import jax.numpy as jnp
from jax.experimental import pallas as pl


def kernel(x, m_list, target):
    raise NotImplementedError("write your pallas kernel here")


# speedup vs baseline: 3.3873x; 3.3873x over previous
import functools

import jax
import jax.numpy as jnp
from jax import lax
from jax.experimental import pallas as pl

_S = 30.0


def _ldam_body(x_ref, tgt_ref, ml_ref, out_ref, *, nrows_total):
    xb = x_ref[...]
    tgt = tgt_ref[...]
    ml = ml_ref[...]
    blk, c = xb.shape
    col = lax.broadcasted_iota(jnp.int32, (blk, c), 1)
    onehot = col == tgt
    m_row = jnp.sum(jnp.where(onehot, ml, 0.0), axis=1, keepdims=True)
    logits = _S * jnp.where(onehot, xb - m_row, xb)
    mx = jnp.max(logits, axis=1, keepdims=True)
    se = jnp.sum(jnp.exp(logits - mx), axis=1, keepdims=True)
    logz = jnp.log(se) + mx
    tgt_logit = jnp.sum(jnp.where(onehot, logits, 0.0), axis=1, keepdims=True)
    part = (jnp.sum(logz - tgt_logit) * (1.0 / nrows_total)).reshape(1, 1)

    @pl.when(pl.program_id(0) == 0)
    def _():
        out_ref[...] = jnp.zeros_like(out_ref)

    out_ref[...] += part


def kernel(x, m_list, target):
    b, c = x.shape
    blk = 1024
    grid = b // blk
    out = pl.pallas_call(
        functools.partial(_ldam_body, nrows_total=b),
        grid=(grid,),
        in_specs=[
            pl.BlockSpec((blk, c), lambda i: (i, 0)),
            pl.BlockSpec((blk, 1), lambda i: (i, 0)),
            pl.BlockSpec((1, c), lambda i: (0, 0)),
        ],
        out_specs=pl.BlockSpec((1, 1), lambda i: (0, 0)),
        out_shape=jax.ShapeDtypeStruct((1, 1), jnp.float32),
    )(x, target.reshape(b, 1), m_list.reshape(1, c))
    return out[0, 0]


# speedup vs baseline: 3.8292x; 1.1304x over previous
import functools

import jax
import jax.numpy as jnp
from jax import lax
from jax.experimental import pallas as pl

_S = 30.0


def _ldam_body(x_ref, tgt_ref, ml_ref, out_ref, *, nrows_total):
    xb = x_ref[...]
    tgt = tgt_ref[...]
    ml = ml_ref[...]
    blk, c = xb.shape
    col = lax.broadcasted_iota(jnp.int32, (blk, c), 1)
    onehot = col == tgt
    m_row = jnp.sum(jnp.where(onehot, ml, 0.0), axis=1, keepdims=True)
    logits = _S * jnp.where(onehot, xb - m_row, xb)
    mx = jnp.max(logits, axis=1, keepdims=True)
    se = jnp.sum(jnp.exp(logits - mx), axis=1, keepdims=True)
    logz = jnp.log(se) + mx
    tgt_logit = jnp.sum(jnp.where(onehot, logits, 0.0), axis=1, keepdims=True)
    part = (jnp.sum(logz - tgt_logit) * (1.0 / nrows_total)).reshape(1, 1)

    @pl.when(pl.program_id(0) == 0)
    def _():
        out_ref[...] = jnp.zeros_like(out_ref)

    out_ref[...] += part


def kernel(x, m_list, target):
    b, c = x.shape
    blk = 2048
    grid = b // blk
    out = pl.pallas_call(
        functools.partial(_ldam_body, nrows_total=b),
        grid=(grid,),
        in_specs=[
            pl.BlockSpec((blk, c), lambda i: (i, 0)),
            pl.BlockSpec((blk, 1), lambda i: (i, 0)),
            pl.BlockSpec((1, c), lambda i: (0, 0)),
        ],
        out_specs=pl.BlockSpec((1, 1), lambda i: (0, 0)),
        out_shape=jax.ShapeDtypeStruct((1, 1), jnp.float32),
    )(x, target.reshape(b, 1), m_list.reshape(1, c))
    return out[0, 0]


# speedup vs baseline: 3.8641x; 1.0091x over previous
import functools

import jax
import jax.numpy as jnp
from jax import lax
from jax.experimental import pallas as pl

_S = 30.0


def _ldam_body(x_ref, tgt_ref, ml_ref, out_ref, *, nrows_total):
    xb = x_ref[...]
    tgt = tgt_ref[...]
    ml = ml_ref[...]
    blk, c = xb.shape
    col = lax.broadcasted_iota(jnp.int32, (blk, c), 1)
    onehot = col == tgt
    m_row = jnp.sum(jnp.where(onehot, ml, 0.0), axis=1, keepdims=True)
    logits = _S * jnp.where(onehot, xb - m_row, xb)
    mx = jnp.max(logits, axis=1, keepdims=True)
    se = jnp.sum(jnp.exp(logits - mx), axis=1, keepdims=True)
    logz = jnp.log(se) + mx
    tgt_logit = jnp.sum(jnp.where(onehot, logits, 0.0), axis=1, keepdims=True)
    part = (jnp.sum(logz - tgt_logit) * (1.0 / nrows_total)).reshape(1, 1)

    @pl.when(pl.program_id(0) == 0)
    def _():
        out_ref[...] = jnp.zeros_like(out_ref)

    out_ref[...] += part


def kernel(x, m_list, target):
    b, c = x.shape
    blk = 4096
    grid = b // blk
    out = pl.pallas_call(
        functools.partial(_ldam_body, nrows_total=b),
        grid=(grid,),
        in_specs=[
            pl.BlockSpec((blk, c), lambda i: (i, 0)),
            pl.BlockSpec((blk, 1), lambda i: (i, 0)),
            pl.BlockSpec((1, c), lambda i: (0, 0)),
        ],
        out_specs=pl.BlockSpec((1, 1), lambda i: (0, 0)),
        out_shape=jax.ShapeDtypeStruct((1, 1), jnp.float32),
    )(x, target.reshape(b, 1), m_list.reshape(1, c))
    return out[0, 0]
